# baseline (speedup 1.0000x reference)
import jax
import jax.numpy as jnp
from jax.experimental import pallas as pl
from jax.experimental.pallas import tpu as pltpu

L = 50
OUT = 10
G = 8
TB = 16384


def _gate_kernel(x_ref, w_ref, b_ref, s_ref, o_ref):
    m = x_ref.shape[0]
    xr = x_ref[...].reshape(m, G * L)
    y = jnp.dot(xr, w_ref[...], preferred_element_type=jnp.float32)
    y = jax.nn.sigmoid(y + b_ref[...])
    e = jnp.exp(y)
    denom = jnp.dot(e, s_ref[...], preferred_element_type=jnp.float32)
    o_ref[...] = e * pl.reciprocal(denom, approx=True)


def kernel(x, w_fused, b_fused):
    B = x.shape[0]
    assert x.shape[1] == 1 and x.shape[2] == L
    x = x.astype(jnp.float32)
    w_fused = w_fused.astype(jnp.float32)
    b_fused = b_fused.astype(jnp.float32)

    tb = B if B <= TB else TB
    grid = (pl.cdiv(B, tb),)

    x3 = x.reshape(B // G, G, L)

    eye = jnp.eye(G, dtype=jnp.float32)
    w_big = jnp.kron(eye, w_fused)
    b_big = jnp.tile(b_fused, (1, G))
    s_big = jnp.kron(eye, jnp.ones((OUT, OUT), jnp.float32))

    out = pl.pallas_call(
        _gate_kernel,
        out_shape=jax.ShapeDtypeStruct((B // G, G * OUT), jnp.float32),
        grid=grid,
        in_specs=[
            pl.BlockSpec((tb // G, G, L), lambda i: (i, 0, 0)),
            pl.BlockSpec((G * L, G * OUT), lambda i: (0, 0)),
            pl.BlockSpec((1, G * OUT), lambda i: (0, 0)),
            pl.BlockSpec((G * OUT, G * OUT), lambda i: (0, 0)),
        ],
        out_specs=pl.BlockSpec((tb // G, G * OUT), lambda i: (i, 0)),
        compiler_params=pltpu.CompilerParams(
            dimension_semantics=("parallel",)),
    )(x3, w_big, b_big, s_big)

    return out.reshape(B, 1, OUT)
